# baseline (speedup 1.0000x reference)
import functools

import jax
import jax.numpy as jnp
from jax import lax
from jax.experimental import pallas as pl
from jax.experimental.pallas import tpu as pltpu
from jax.experimental.pallas import tpu_sc as plsc

K = 16
P = 20000
PSC = 20096
PPAD = 20096
QTOT = 4096
TQ = 256


def _topk_tc_body(a_ref, b_ref, d_ref, i_ref):
    scores = jnp.dot(
        a_ref[...], b_ref[...],
        precision=lax.Precision.HIGHEST,
        preferred_element_type=jnp.float32,
    )
    col = lax.broadcasted_iota(jnp.int32, (TQ, PPAD), 1)
    d_list, i_list = [], []
    m = jnp.full((TQ, 1), -jnp.inf, jnp.float32)
    am = jnp.full((TQ, 1), -1, jnp.int32)
    for _ in range(K):
        gt = (scores > m) | ((scores == m) & (col > am))
        m_new = jnp.min(jnp.where(gt, scores, jnp.float32(jnp.inf)),
                        axis=1, keepdims=True)
        tcol = jnp.where(m_new == m, am, -1)
        am = jnp.min(
            jnp.where((scores == m_new) & (col > tcol), col, PPAD),
            axis=1, keepdims=True)
        m = m_new
        d_list.append(m)
        i_list.append(am)
    d2 = jnp.concatenate(d_list, axis=1)
    d_ref[...] = jnp.sqrt(jnp.maximum(d2, 1e-12))
    i_ref[...] = jnp.concatenate(i_list, axis=1)


def _topk_tc(a, b):
    return pl.pallas_call(
        _topk_tc_body,
        grid=(QTOT // TQ,),
        in_specs=[
            pl.BlockSpec((TQ, 8), lambda i: (i, 0)),
            pl.BlockSpec((8, PPAD), lambda i: (0, 0)),
        ],
        out_specs=[
            pl.BlockSpec((TQ, K), lambda i: (i, 0)),
            pl.BlockSpec((TQ, K), lambda i: (i, 0)),
        ],
        out_shape=[
            jax.ShapeDtypeStruct((QTOT, K), jnp.float32),
            jax.ShapeDtypeStruct((QTOT, K), jnp.int32),
        ],
    )(a, b)


def _combine_sc(idx_t, dd_t, q_t, nx, ny, nz, pn, cf):
    info = plsc.get_sparse_core_info()
    nc, ns = info.num_cores, info.num_subcores
    nw = nc * ns
    nblk = QTOT // 16
    bw = nblk // nw

    mesh = plsc.VectorSubcoreMesh(core_axis_name="c", subcore_axis_name="s")

    @functools.partial(
        pl.kernel,
        mesh=mesh,
        compiler_params=pltpu.CompilerParams(
            use_tc_tiling_on_sc=False, needs_layout_passes=False
        ),
        out_type=jax.ShapeDtypeStruct((nblk, 16), jnp.float32),
        scratch_types=[
            pltpu.VMEM((PSC,), jnp.float32),
            pltpu.VMEM((PSC,), jnp.float32),
            pltpu.VMEM((PSC,), jnp.float32),
            pltpu.VMEM((PSC,), jnp.float32),
            pltpu.VMEM((PSC,), jnp.float32),
            pltpu.VMEM((bw, K, 16), jnp.int32),
            pltpu.VMEM((bw, K, 16), jnp.float32),
            pltpu.VMEM((bw, 3, 16), jnp.float32),
            pltpu.VMEM((bw, 16), jnp.float32),
        ],
    )
    def sc_kernel(idx_h, dd_h, q_h, nx_h, ny_h, nz_h, pn_h, cf_h, out_h,
                  nx_v, ny_v, nz_v, pn_v, cf_v, idx_v, dd_v, q_v, out_v):
        wid = lax.axis_index("s") * nc + lax.axis_index("c")
        base = wid * bw
        pltpu.sync_copy(nx_h, nx_v)
        pltpu.sync_copy(ny_h, ny_v)
        pltpu.sync_copy(nz_h, nz_v)
        pltpu.sync_copy(pn_h, pn_v)
        pltpu.sync_copy(cf_h, cf_v)
        pltpu.sync_copy(idx_h.at[pl.ds(base, bw)], idx_v)
        pltpu.sync_copy(dd_h.at[pl.ds(base, bw)], dd_v)
        pltpu.sync_copy(q_h.at[pl.ds(base, bw)], q_v)

        def body(blk, carry):
            qx = q_v[blk, 0]
            qy = q_v[blk, 1]
            qz = q_v[blk, 2]
            numer = jnp.zeros((16,), jnp.float32)
            denom = jnp.zeros((16,), jnp.float32)
            for k in range(K):
                iv = idx_v[blk, k]
                dv = dd_v[blk, k]
                nxg = plsc.load_gather(nx_v, [iv])
                nyg = plsc.load_gather(ny_v, [iv])
                nzg = plsc.load_gather(nz_v, [iv])
                png = plsc.load_gather(pn_v, [iv])
                cg = plsc.load_gather(cf_v, [iv])
                qdotn = qx * nxg + qy * nyg + qz * nzg
                sg = jnp.sign(png - qdotn)
                numer = numer + sg * cg
                denom = denom + cg / dv
            out_v[blk] = numer / denom
            return carry

        lax.fori_loop(0, bw, body, 0)
        pltpu.sync_copy(out_v, out_h.at[pl.ds(base, bw)])

    return sc_kernel(idx_t, dd_t, q_t, nx, ny, nz, pn, cf)


def kernel(pts, points, normals, conf):
    q = pts.reshape(QTOT, 3)
    q2 = jnp.sum(q * q, axis=1, keepdims=True)
    a = jnp.concatenate(
        [q, q2, jnp.ones((QTOT, 1), jnp.float32), jnp.zeros((QTOT, 3), jnp.float32)],
        axis=1,
    )
    p2 = jnp.sum(points * points, axis=1)
    pn = jnp.sum(points * normals, axis=1)
    b = jnp.zeros((8, PPAD), jnp.float32)
    b = b.at[0:3, :P].set(-2.0 * points.T)
    b = b.at[3, :].set(1.0)
    b = b.at[4, :P].set(p2)
    b = b.at[4, P:].set(1e30)

    dd, idx = _topk_tc(a, b)

    nblk = QTOT // 16
    idx_t = idx.reshape(nblk, 16, K).transpose(0, 2, 1)
    dd_t = dd.reshape(nblk, 16, K).transpose(0, 2, 1)
    q_t = q.reshape(nblk, 16, 3).transpose(0, 2, 1)

    pad = (0, PSC - P)
    out = _combine_sc(
        idx_t, dd_t, q_t,
        jnp.pad(normals[:, 0], pad), jnp.pad(normals[:, 1], pad),
        jnp.pad(normals[:, 2], pad), jnp.pad(pn, pad),
        jnp.pad(conf[:, 0], pad, constant_values=1.0),
    )
    return out.reshape(pts.shape[0], pts.shape[1])


# speedup vs baseline: 1.5859x; 1.5859x over previous
import functools

import jax
import jax.numpy as jnp
from jax import lax
from jax.experimental import pallas as pl
from jax.experimental.pallas import tpu as pltpu
from jax.experimental.pallas import tpu_sc as plsc

K = 16
P = 20000
PSC = 20096
PPAD = 20096
QTOT = 4096
TQ = 256


def _topk_tc_body(a_ref, b_ref, d_ref, i_ref):
    scores = jnp.dot(
        a_ref[...], b_ref[...],
        precision=lax.Precision.HIGHEST,
        preferred_element_type=jnp.float32,
    )
    col = lax.broadcasted_iota(jnp.int32, (TQ, PPAD), 1)
    d_list, i_list = [], []
    m = jnp.min(scores, axis=1, keepdims=True)
    am = jnp.argmin(scores, axis=1).astype(jnp.int32)[:, None]
    d_list.append(m)
    i_list.append(am)
    for _ in range(K - 1):
        scores = jnp.where(col == am, jnp.float32(jnp.inf), scores)
        m = jnp.min(scores, axis=1, keepdims=True)
        am = jnp.argmin(scores, axis=1).astype(jnp.int32)[:, None]
        d_list.append(m)
        i_list.append(am)
    d2 = jnp.concatenate(d_list, axis=1)
    d_ref[...] = jnp.sqrt(jnp.maximum(d2, 1e-12))
    i_ref[...] = jnp.concatenate(i_list, axis=1)


def _topk_tc(a, b):
    return pl.pallas_call(
        _topk_tc_body,
        grid=(QTOT // TQ,),
        in_specs=[
            pl.BlockSpec((TQ, 8), lambda i: (i, 0)),
            pl.BlockSpec((8, PPAD), lambda i: (0, 0)),
        ],
        out_specs=[
            pl.BlockSpec((TQ, K), lambda i: (i, 0)),
            pl.BlockSpec((TQ, K), lambda i: (i, 0)),
        ],
        out_shape=[
            jax.ShapeDtypeStruct((QTOT, K), jnp.float32),
            jax.ShapeDtypeStruct((QTOT, K), jnp.int32),
        ],
    )(a, b)


def _combine_sc(idx_t, dd_t, q_t, nx, ny, nz, pn, cf):
    info = plsc.get_sparse_core_info()
    nc, ns = info.num_cores, info.num_subcores
    nw = nc * ns
    nblk = QTOT // 16
    bw = nblk // nw

    mesh = plsc.VectorSubcoreMesh(core_axis_name="c", subcore_axis_name="s")

    @functools.partial(
        pl.kernel,
        mesh=mesh,
        compiler_params=pltpu.CompilerParams(
            use_tc_tiling_on_sc=False, needs_layout_passes=False
        ),
        out_type=jax.ShapeDtypeStruct((nblk, 16), jnp.float32),
        scratch_types=[
            pltpu.VMEM((PSC,), jnp.float32),
            pltpu.VMEM((PSC,), jnp.float32),
            pltpu.VMEM((PSC,), jnp.float32),
            pltpu.VMEM((PSC,), jnp.float32),
            pltpu.VMEM((PSC,), jnp.float32),
            pltpu.VMEM((bw, K, 16), jnp.int32),
            pltpu.VMEM((bw, K, 16), jnp.float32),
            pltpu.VMEM((bw, 3, 16), jnp.float32),
            pltpu.VMEM((bw, 16), jnp.float32),
        ],
    )
    def sc_kernel(idx_h, dd_h, q_h, nx_h, ny_h, nz_h, pn_h, cf_h, out_h,
                  nx_v, ny_v, nz_v, pn_v, cf_v, idx_v, dd_v, q_v, out_v):
        wid = lax.axis_index("s") * nc + lax.axis_index("c")
        base = wid * bw
        pltpu.sync_copy(nx_h, nx_v)
        pltpu.sync_copy(ny_h, ny_v)
        pltpu.sync_copy(nz_h, nz_v)
        pltpu.sync_copy(pn_h, pn_v)
        pltpu.sync_copy(cf_h, cf_v)
        pltpu.sync_copy(idx_h.at[pl.ds(base, bw)], idx_v)
        pltpu.sync_copy(dd_h.at[pl.ds(base, bw)], dd_v)
        pltpu.sync_copy(q_h.at[pl.ds(base, bw)], q_v)

        def body(blk, carry):
            qx = q_v[blk, 0]
            qy = q_v[blk, 1]
            qz = q_v[blk, 2]
            numer = jnp.zeros((16,), jnp.float32)
            denom = jnp.zeros((16,), jnp.float32)
            for k in range(K):
                iv = idx_v[blk, k]
                dv = dd_v[blk, k]
                nxg = plsc.load_gather(nx_v, [iv])
                nyg = plsc.load_gather(ny_v, [iv])
                nzg = plsc.load_gather(nz_v, [iv])
                png = plsc.load_gather(pn_v, [iv])
                cg = plsc.load_gather(cf_v, [iv])
                qdotn = qx * nxg + qy * nyg + qz * nzg
                sg = jnp.sign(png - qdotn)
                numer = numer + sg * cg
                denom = denom + cg / dv
            out_v[blk] = numer / denom
            return carry

        lax.fori_loop(0, bw, body, 0)
        pltpu.sync_copy(out_v, out_h.at[pl.ds(base, bw)])

    return sc_kernel(idx_t, dd_t, q_t, nx, ny, nz, pn, cf)


def kernel(pts, points, normals, conf):
    q = pts.reshape(QTOT, 3)
    q2 = jnp.sum(q * q, axis=1, keepdims=True)
    a = jnp.concatenate(
        [q, q2, jnp.ones((QTOT, 1), jnp.float32), jnp.zeros((QTOT, 3), jnp.float32)],
        axis=1,
    )
    p2 = jnp.sum(points * points, axis=1)
    pn = jnp.sum(points * normals, axis=1)
    b = jnp.zeros((8, PPAD), jnp.float32)
    b = b.at[0:3, :P].set(-2.0 * points.T)
    b = b.at[3, :].set(1.0)
    b = b.at[4, :P].set(p2)
    b = b.at[4, P:].set(1e30)

    dd, idx = _topk_tc(a, b)

    nblk = QTOT // 16
    idx_t = idx.reshape(nblk, 16, K).transpose(0, 2, 1)
    dd_t = dd.reshape(nblk, 16, K).transpose(0, 2, 1)
    q_t = q.reshape(nblk, 16, 3).transpose(0, 2, 1)

    pad = (0, PSC - P)
    out = _combine_sc(
        idx_t, dd_t, q_t,
        jnp.pad(normals[:, 0], pad), jnp.pad(normals[:, 1], pad),
        jnp.pad(normals[:, 2], pad), jnp.pad(pn, pad),
        jnp.pad(conf[:, 0], pad, constant_values=1.0),
    )
    return out.reshape(pts.shape[0], pts.shape[1])


# speedup vs baseline: 1.6510x; 1.0410x over previous
import functools

import jax
import jax.numpy as jnp
from jax import lax
from jax.experimental import pallas as pl
from jax.experimental.pallas import tpu as pltpu
from jax.experimental.pallas import tpu_sc as plsc

K = 16
P = 20000
PSC = 20096
F = 8
G = 2560
PPAD = F * G
QTOT = 4096
TQ = 128

_SORT8 = [
    (0, 1), (2, 3), (4, 5), (6, 7),
    (0, 2), (1, 3), (4, 6), (5, 7),
    (1, 2), (5, 6),
    (0, 4), (1, 5), (2, 6), (3, 7),
    (2, 4), (3, 5),
    (1, 2), (3, 4), (5, 6),
]


def _topk_tc_body(a_ref, b_ref, d_ref, i_ref):
    scores = jnp.dot(
        a_ref[...], b_ref[...],
        precision=lax.Precision.HIGHEST,
        preferred_element_type=jnp.float32,
    )
    colg = lax.broadcasted_iota(jnp.int32, (TQ, G), 1)
    v = [scores[:, l * G:(l + 1) * G] for l in range(F)]
    ix = [colg + l * G for l in range(F)]
    for a, b in _SORT8:
        sw = (v[a] > v[b]) | ((v[a] == v[b]) & (ix[a] > ix[b]))
        va = jnp.where(sw, v[b], v[a])
        vb = jnp.where(sw, v[a], v[b])
        ia = jnp.where(sw, ix[b], ix[a])
        ib = jnp.where(sw, ix[a], ix[b])
        v[a], v[b], ix[a], ix[b] = va, vb, ia, ib

    d_list, i_list = [], []
    big = jnp.int32(0x7FFFFFFF)
    for _ in range(K):
        m = jnp.min(v[0], axis=1, keepdims=True)
        am = jnp.min(jnp.where(v[0] == m, ix[0], big), axis=1, keepdims=True)
        p = (v[0] == m) & (ix[0] == am)
        i_list.append(am)
        d_list.append(m)
        for l in range(F - 1):
            v[l] = jnp.where(p, v[l + 1], v[l])
            ix[l] = jnp.where(p, ix[l + 1], ix[l])
        v[F - 1] = jnp.where(p, jnp.float32(jnp.inf), v[F - 1])
    d2 = jnp.concatenate(d_list, axis=1)
    d_ref[...] = jnp.sqrt(jnp.maximum(d2, 1e-12))
    i_ref[...] = jnp.concatenate(i_list, axis=1)


def _topk_tc(a, b):
    return pl.pallas_call(
        _topk_tc_body,
        grid=(QTOT // TQ,),
        in_specs=[
            pl.BlockSpec((TQ, 8), lambda i: (i, 0)),
            pl.BlockSpec((8, PPAD), lambda i: (0, 0)),
        ],
        out_specs=[
            pl.BlockSpec((TQ, K), lambda i: (i, 0)),
            pl.BlockSpec((TQ, K), lambda i: (i, 0)),
        ],
        out_shape=[
            jax.ShapeDtypeStruct((QTOT, K), jnp.float32),
            jax.ShapeDtypeStruct((QTOT, K), jnp.int32),
        ],
    )(a, b)


def _combine_sc(idx_t, dd_t, q_t, nx, ny, nz, pn, cf):
    info = plsc.get_sparse_core_info()
    nc, ns = info.num_cores, info.num_subcores
    nw = nc * ns
    nblk = QTOT // 16
    bw = nblk // nw

    mesh = plsc.VectorSubcoreMesh(core_axis_name="c", subcore_axis_name="s")

    @functools.partial(
        pl.kernel,
        mesh=mesh,
        compiler_params=pltpu.CompilerParams(
            use_tc_tiling_on_sc=False, needs_layout_passes=False
        ),
        out_type=jax.ShapeDtypeStruct((nblk, 16), jnp.float32),
        scratch_types=[
            pltpu.VMEM((PSC,), jnp.float32),
            pltpu.VMEM((PSC,), jnp.float32),
            pltpu.VMEM((PSC,), jnp.float32),
            pltpu.VMEM((PSC,), jnp.float32),
            pltpu.VMEM((PSC,), jnp.float32),
            pltpu.VMEM((bw, K, 16), jnp.int32),
            pltpu.VMEM((bw, K, 16), jnp.float32),
            pltpu.VMEM((bw, 3, 16), jnp.float32),
            pltpu.VMEM((bw, 16), jnp.float32),
        ],
    )
    def sc_kernel(idx_h, dd_h, q_h, nx_h, ny_h, nz_h, pn_h, cf_h, out_h,
                  nx_v, ny_v, nz_v, pn_v, cf_v, idx_v, dd_v, q_v, out_v):
        wid = lax.axis_index("s") * nc + lax.axis_index("c")
        base = wid * bw
        pltpu.sync_copy(nx_h, nx_v)
        pltpu.sync_copy(ny_h, ny_v)
        pltpu.sync_copy(nz_h, nz_v)
        pltpu.sync_copy(pn_h, pn_v)
        pltpu.sync_copy(cf_h, cf_v)
        pltpu.sync_copy(idx_h.at[pl.ds(base, bw)], idx_v)
        pltpu.sync_copy(dd_h.at[pl.ds(base, bw)], dd_v)
        pltpu.sync_copy(q_h.at[pl.ds(base, bw)], q_v)

        def body(blk, carry):
            qx = q_v[blk, 0]
            qy = q_v[blk, 1]
            qz = q_v[blk, 2]
            numer = jnp.zeros((16,), jnp.float32)
            denom = jnp.zeros((16,), jnp.float32)
            for k in range(K):
                iv = idx_v[blk, k]
                dv = dd_v[blk, k]
                nxg = plsc.load_gather(nx_v, [iv])
                nyg = plsc.load_gather(ny_v, [iv])
                nzg = plsc.load_gather(nz_v, [iv])
                png = plsc.load_gather(pn_v, [iv])
                cg = plsc.load_gather(cf_v, [iv])
                qdotn = qx * nxg + qy * nyg + qz * nzg
                sg = jnp.sign(png - qdotn)
                numer = numer + sg * cg
                denom = denom + cg / dv
            out_v[blk] = numer / denom
            return carry

        lax.fori_loop(0, bw, body, 0)
        pltpu.sync_copy(out_v, out_h.at[pl.ds(base, bw)])

    return sc_kernel(idx_t, dd_t, q_t, nx, ny, nz, pn, cf)


def kernel(pts, points, normals, conf):
    q = pts.reshape(QTOT, 3)
    q2 = jnp.sum(q * q, axis=1, keepdims=True)
    a = jnp.concatenate(
        [q, q2, jnp.ones((QTOT, 1), jnp.float32), jnp.zeros((QTOT, 3), jnp.float32)],
        axis=1,
    )
    p2 = jnp.sum(points * points, axis=1)
    pn = jnp.sum(points * normals, axis=1)
    b = jnp.zeros((8, PPAD), jnp.float32)
    b = b.at[0:3, :P].set(-2.0 * points.T)
    b = b.at[3, :].set(1.0)
    b = b.at[4, :P].set(p2)
    b = b.at[4, P:].set(1e30)

    dd, idx = _topk_tc(a, b)

    nblk = QTOT // 16
    idx_t = idx.reshape(nblk, 16, K).transpose(0, 2, 1)
    dd_t = dd.reshape(nblk, 16, K).transpose(0, 2, 1)
    q_t = q.reshape(nblk, 16, 3).transpose(0, 2, 1)

    pad = (0, PSC - P)
    out = _combine_sc(
        idx_t, dd_t, q_t,
        jnp.pad(normals[:, 0], pad), jnp.pad(normals[:, 1], pad),
        jnp.pad(normals[:, 2], pad), jnp.pad(pn, pad),
        jnp.pad(conf[:, 0], pad, constant_values=1.0),
    )
    return out.reshape(pts.shape[0], pts.shape[1])


# speedup vs baseline: 1.6517x; 1.0004x over previous
import functools

import jax
import jax.numpy as jnp
from jax import lax
from jax.experimental import pallas as pl
from jax.experimental.pallas import tpu as pltpu
from jax.experimental.pallas import tpu_sc as plsc

K = 16
P = 20000
PSC = 20096
F = 8
G = 2560
PPAD = F * G
QTOT = 4096
TQ = 128

_SORT8 = [
    (0, 1), (2, 3), (4, 5), (6, 7),
    (0, 2), (1, 3), (4, 6), (5, 7),
    (1, 2), (5, 6),
    (0, 4), (1, 5), (2, 6), (3, 7),
    (2, 4), (3, 5),
    (1, 2), (3, 4), (5, 6),
]


def _topk_tc_body(a_ref, b_ref, d_ref, i_ref):
    scores = jnp.dot(
        a_ref[...], b_ref[...],
        precision=lax.Precision.HIGHEST,
        preferred_element_type=jnp.float32,
    )
    colg = lax.broadcasted_iota(jnp.int32, (TQ, G), 1)
    v = [scores[:, l * G:(l + 1) * G] for l in range(F)]
    ix = [colg + l * G for l in range(F)]
    for a, b in _SORT8:
        sw = (v[a] > v[b]) | ((v[a] == v[b]) & (ix[a] > ix[b]))
        va = jnp.where(sw, v[b], v[a])
        vb = jnp.where(sw, v[a], v[b])
        ia = jnp.where(sw, ix[b], ix[a])
        ib = jnp.where(sw, ix[a], ix[b])
        v[a], v[b], ix[a], ix[b] = va, vb, ia, ib

    d_list, i_list = [], []
    big = jnp.int32(0x7FFFFFFF)
    for k in range(K):
        m = jnp.min(v[0], axis=1, keepdims=True)
        am = jnp.min(jnp.where(v[0] == m, ix[0], big), axis=1, keepdims=True)
        i_list.append(am)
        d_list.append(m)
        if k < K - 1:
            p = (v[0] == m) & (ix[0] == am)
            for l in range(F - 1):
                v[l] = jnp.where(p, v[l + 1], v[l])
                ix[l] = jnp.where(p, ix[l + 1], ix[l])
            v[F - 1] = jnp.where(p, jnp.float32(jnp.inf), v[F - 1])
    d2 = jnp.concatenate(d_list, axis=1)
    d_ref[...] = jnp.sqrt(jnp.maximum(d2, 1e-12))
    i_ref[...] = jnp.concatenate(i_list, axis=1)


def _topk_tc(a, b):
    return pl.pallas_call(
        _topk_tc_body,
        grid=(QTOT // TQ,),
        in_specs=[
            pl.BlockSpec((TQ, 8), lambda i: (i, 0)),
            pl.BlockSpec((8, PPAD), lambda i: (0, 0)),
        ],
        out_specs=[
            pl.BlockSpec((TQ, K), lambda i: (i, 0)),
            pl.BlockSpec((TQ, K), lambda i: (i, 0)),
        ],
        out_shape=[
            jax.ShapeDtypeStruct((QTOT, K), jnp.float32),
            jax.ShapeDtypeStruct((QTOT, K), jnp.int32),
        ],
    )(a, b)


def _combine_sc(idx_t, dd_t, q_t, nx, ny, nz, pn, cf):
    info = plsc.get_sparse_core_info()
    nc, ns = info.num_cores, info.num_subcores
    nw = nc * ns
    nblk = QTOT // 16
    bw = nblk // nw

    mesh = plsc.VectorSubcoreMesh(core_axis_name="c", subcore_axis_name="s")

    @functools.partial(
        pl.kernel,
        mesh=mesh,
        compiler_params=pltpu.CompilerParams(
            use_tc_tiling_on_sc=False, needs_layout_passes=False
        ),
        out_type=jax.ShapeDtypeStruct((nblk, 16), jnp.float32),
        scratch_types=[
            pltpu.VMEM((PSC,), jnp.float32),
            pltpu.VMEM((PSC,), jnp.float32),
            pltpu.VMEM((PSC,), jnp.float32),
            pltpu.VMEM((PSC,), jnp.float32),
            pltpu.VMEM((PSC,), jnp.float32),
            pltpu.VMEM((bw, K, 16), jnp.int32),
            pltpu.VMEM((bw, K, 16), jnp.float32),
            pltpu.VMEM((bw, 3, 16), jnp.float32),
            pltpu.VMEM((bw, 16), jnp.float32),
        ],
    )
    def sc_kernel(idx_h, dd_h, q_h, nx_h, ny_h, nz_h, pn_h, cf_h, out_h,
                  nx_v, ny_v, nz_v, pn_v, cf_v, idx_v, dd_v, q_v, out_v):
        wid = lax.axis_index("s") * nc + lax.axis_index("c")
        base = wid * bw
        pltpu.sync_copy(nx_h, nx_v)
        pltpu.sync_copy(ny_h, ny_v)
        pltpu.sync_copy(nz_h, nz_v)
        pltpu.sync_copy(pn_h, pn_v)
        pltpu.sync_copy(cf_h, cf_v)
        pltpu.sync_copy(idx_h.at[pl.ds(base, bw)], idx_v)
        pltpu.sync_copy(dd_h.at[pl.ds(base, bw)], dd_v)
        pltpu.sync_copy(q_h.at[pl.ds(base, bw)], q_v)

        def body(blk, carry):
            qx = q_v[blk, 0]
            qy = q_v[blk, 1]
            qz = q_v[blk, 2]
            numer = jnp.zeros((16,), jnp.float32)
            denom = jnp.zeros((16,), jnp.float32)
            for k in range(K):
                iv = idx_v[blk, k]
                dv = dd_v[blk, k]
                nxg = plsc.load_gather(nx_v, [iv])
                nyg = plsc.load_gather(ny_v, [iv])
                nzg = plsc.load_gather(nz_v, [iv])
                png = plsc.load_gather(pn_v, [iv])
                cg = plsc.load_gather(cf_v, [iv])
                qdotn = qx * nxg + qy * nyg + qz * nzg
                sg = jnp.sign(png - qdotn)
                numer = numer + sg * cg
                denom = denom + cg / dv
            out_v[blk] = numer / denom
            return carry

        lax.fori_loop(0, bw, body, 0)
        pltpu.sync_copy(out_v, out_h.at[pl.ds(base, bw)])

    return sc_kernel(idx_t, dd_t, q_t, nx, ny, nz, pn, cf)


def kernel(pts, points, normals, conf):
    q = pts.reshape(QTOT, 3)
    q2 = jnp.sum(q * q, axis=1, keepdims=True)
    a = jnp.concatenate(
        [q, q2, jnp.ones((QTOT, 1), jnp.float32), jnp.zeros((QTOT, 3), jnp.float32)],
        axis=1,
    )
    p2 = jnp.sum(points * points, axis=1)
    pn = jnp.sum(points * normals, axis=1)
    b = jnp.zeros((8, PPAD), jnp.float32)
    b = b.at[0:3, :P].set(-2.0 * points.T)
    b = b.at[3, :].set(1.0)
    b = b.at[4, :P].set(p2)
    b = b.at[4, P:].set(1e30)

    dd, idx = _topk_tc(a, b)

    nblk = QTOT // 16
    idx_t = idx.reshape(nblk, 16, K).transpose(0, 2, 1)
    dd_t = dd.reshape(nblk, 16, K).transpose(0, 2, 1)
    q_t = q.reshape(nblk, 16, 3).transpose(0, 2, 1)

    pad = (0, PSC - P)
    out = _combine_sc(
        idx_t, dd_t, q_t,
        jnp.pad(normals[:, 0], pad), jnp.pad(normals[:, 1], pad),
        jnp.pad(normals[:, 2], pad), jnp.pad(pn, pad),
        jnp.pad(conf[:, 0], pad, constant_values=1.0),
    )
    return out.reshape(pts.shape[0], pts.shape[1])
